# baseline (speedup 1.0000x reference)
import functools

import jax
import jax.numpy as jnp
from jax.experimental import pallas as pl
from jax.experimental.pallas import tpu as pltpu

_LANE = 128
_R = 32


def _ceil_to(a, b):
    return (a + b - 1) // b * b


def _pack_bf16_pair(c0, c1):
    u0 = jax.lax.bitcast_convert_type(c0, jnp.uint32)
    u1 = jax.lax.bitcast_convert_type(c1, jnp.uint32)
    one = jnp.uint32(1)
    r0 = (u0 + jnp.uint32(0x7FFF) + ((u0 >> 16) & one)) >> 16
    r1 = (u1 + jnp.uint32(0x7FFF) + ((u1 >> 16) & one)) & jnp.uint32(0xFFFF0000)
    return jax.lax.bitcast_convert_type(r0 | r1, jnp.int32)


def _gather_batch(idx_ref, scl_ref, src_ref, K, pi, r0):
    accs = []
    for r in range(_R):
        acc = None
        for k in range(K):
            i = pl.multiple_of(idx_ref[k, r0 + r], pi)
            slab = pltpu.bitcast(src_ref[pl.ds(i, pi), :],
                                 jnp.bfloat16).astype(jnp.float32)
            acc = slab if acc is None else acc + slab
        accs.append(acc * scl_ref[0, 0, r0 + r])
    return accs


def _edge_body(K, t_rows, p, idx_ref, scl_ref, xp_ref, w2_ref, e1_ref, scr):
    pi = p // 2
    for b in range(t_rows // _R):
        r0 = b * _R
        accs = _gather_batch(idx_ref, scl_ref, xp_ref, K, pi, r0)
        for r in range(_R):
            rr = (r0 + r) * p
            scr[pl.ds(rr, p), :] = jnp.maximum(accs[r], 0.0)

    xs = [scr[pl.Slice(j, t_rows, p), :] for j in range(p)]
    xt = jnp.concatenate(xs, axis=-1) if p > 1 else xs[0]
    res = jnp.dot(xt.astype(jnp.bfloat16), w2_ref[...],
                  preferred_element_type=jnp.float32)
    for q in range(pi):
        e1_ref[pl.Slice(q, t_rows, pi), :] = _pack_bf16_pair(
            res[:, (2 * q) * _LANE:(2 * q + 1) * _LANE],
            res[:, (2 * q + 1) * _LANE:(2 * q + 2) * _LANE])


def _node_body(K, t_rows, p, idx_ref, scl_ref, e1_ref, out_ref, scr):
    pi = p // 2
    for b in range(t_rows // _R):
        r0 = b * _R
        accs = _gather_batch(idx_ref, scl_ref, e1_ref, K, pi, r0)
        for r in range(_R):
            rr = (r0 + r) * p
            scr[pl.ds(rr, p), :] = accs[r]

    xs = [scr[pl.Slice(j, t_rows, p), :] for j in range(p)]
    out_ref[...] = jnp.concatenate(xs, axis=-1) if p > 1 else xs[0]


def _xp_body(p, n_real, inter_ref, x_ref, w1_ref, xp_ref):
    i = pl.program_id(0)
    cn = x_ref.shape[0]

    pi = p // 2

    @pl.when(i < n_real)
    def _():
        res = (jnp.dot(x_ref[...].astype(jnp.bfloat16), w1_ref[...],
                       preferred_element_type=jnp.float32)
               * inter_ref[0, 0])
        for q in range(pi):
            xp_ref[pl.Slice(q, cn, pi), :] = _pack_bf16_pair(
                res[:, (2 * q) * _LANE:(2 * q + 1) * _LANE],
                res[:, (2 * q + 1) * _LANE:(2 * q + 2) * _LANE])

    @pl.when(i >= n_real)
    def _():
        xp_ref[...] = jnp.zeros_like(xp_ref)


def _prep_indices(idx, rows_tot, zrow, p):
    R, K = idx.shape
    f32 = jnp.float32
    it = idx.T.astype(jnp.int32)
    mask = it > 0
    cnt = jnp.sum(mask.astype(jnp.int32), axis=0)
    deg = cnt == 0
    idxr = jnp.where(mask, it, zrow)
    idxr = idxr.at[0].set(jnp.where(deg, 0, idxr[0]))
    scl = jnp.where(deg, 1.0, 1.0 / jnp.maximum(cnt, 1).astype(f32)).astype(f32)
    idx_full = jnp.pad(idxr * p, ((0, 0), (0, rows_tot - R)),
                       constant_values=zrow * p)
    scl_full = jnp.pad(scl, (0, rows_tot - R))
    return idx_full, scl_full


def kernel(x, seq, useq, text_vector, w1, w2, w3):
    del text_vector
    f32 = jnp.float32
    bf16 = jnp.bfloat16
    N, Fin = x.shape
    Fout = w1.shape[1]
    E, K1 = seq.shape
    N2, K2 = useq.shape

    F_pad = _ceil_to(Fout, _LANE)
    p = F_pad // _LANE

    w3f = w3.astype(f32)
    tv = w3f[0]
    cosine = (w3f @ tv) / (jnp.linalg.norm(tv) * jnp.linalg.norm(w3f, axis=1))
    inter = jnp.mean(cosine).reshape(1, 1).astype(f32)

    CN = min(512, _ceil_to(N, _R))
    N_pad = _ceil_to(N, CN)
    N_tot = N_pad + CN
    t_e = min(256, _ceil_to(E, _R))
    E_pad = _ceil_to(E, t_e)
    E_tot = E_pad + t_e
    t_n = min(256, _ceil_to(N2, _R))
    N2_pad = _ceil_to(N2, t_n)

    x_p = x if N == N_pad else jnp.pad(x, ((0, N_pad - N), (0, 0)))
    w1_p = jnp.zeros((Fin, F_pad), bf16).at[:, :Fout].set(w1.astype(bf16))
    w2_p = jnp.zeros((F_pad, F_pad), bf16).at[:Fout, :Fout].set(w2.astype(bf16))

    pi = p // 2
    seq_i, seq_s = _prep_indices(seq, E_tot, N_pad, pi)
    useq_i, useq_s = _prep_indices(useq, N2_pad, E_pad, pi)
    n_et = E_tot // t_e
    n_nt = N2_pad // t_n
    seq_s3 = seq_s.reshape(n_et, 1, t_e)
    useq_s3 = useq_s.reshape(n_nt, 1, t_n)

    smem = pltpu.MemorySpace.SMEM
    vmem_limit = 56 * 1024 * 1024

    def resident(shape):
        return pl.BlockSpec(shape, lambda i: tuple(0 for _ in shape),
                            pipeline_mode=pl.Buffered(1))

    n_real = N_pad // CN
    xp2 = pl.pallas_call(
        functools.partial(_xp_body, p, n_real),
        out_shape=jax.ShapeDtypeStruct((N_tot * pi, _LANE), jnp.int32),
        grid=(N_tot // CN,),
        in_specs=[pl.BlockSpec(memory_space=smem),
                  pl.BlockSpec((CN, Fin),
                               lambda i: (jnp.minimum(i, n_real - 1), 0)),
                  resident((Fin, F_pad))],
        out_specs=pl.BlockSpec((CN * pi, _LANE), lambda i: (i, 0)),
        compiler_params=pltpu.CompilerParams(
            dimension_semantics=("parallel",), vmem_limit_bytes=vmem_limit),
    )(inter, x_p, w1_p)

    e12 = pl.pallas_call(
        functools.partial(_edge_body, K1, t_e, p),
        out_shape=jax.ShapeDtypeStruct((E_tot * pi, _LANE), jnp.int32),
        grid=(n_et,),
        in_specs=[
            pl.BlockSpec((K1, t_e), lambda i: (0, i), memory_space=smem),
            pl.BlockSpec((1, 1, t_e), lambda i: (i, 0, 0), memory_space=smem),
            resident((N_tot * pi, _LANE)),
            resident((F_pad, F_pad)),
        ],
        out_specs=pl.BlockSpec((t_e * pi, _LANE), lambda i: (i, 0)),
        scratch_shapes=[pltpu.VMEM((t_e * p, _LANE), f32)],
        compiler_params=pltpu.CompilerParams(
            dimension_semantics=("parallel",), vmem_limit_bytes=vmem_limit),
    )(seq_i, seq_s3, xp2, w2_p)

    node = pl.pallas_call(
        functools.partial(_node_body, K2, t_n, p),
        out_shape=jax.ShapeDtypeStruct((N2_pad, F_pad), f32),
        grid=(n_nt,),
        in_specs=[
            pl.BlockSpec((K2, t_n), lambda i: (0, i), memory_space=smem),
            pl.BlockSpec((1, 1, t_n), lambda i: (i, 0, 0), memory_space=smem),
            resident((E_tot * pi, _LANE)),
        ],
        out_specs=pl.BlockSpec((t_n, F_pad), lambda i: (i, 0)),
        scratch_shapes=[pltpu.VMEM((t_n * p, _LANE), f32)],
        compiler_params=pltpu.CompilerParams(
            dimension_semantics=("parallel",), vmem_limit_bytes=vmem_limit),
    )(useq_i, useq_s3, e12)

    return node[:N2, :Fout]


# speedup vs baseline: 1.1277x; 1.1277x over previous
import functools

import jax
import jax.numpy as jnp
from jax.experimental import pallas as pl
from jax.experimental.pallas import tpu as pltpu

_LANE = 128
_R = 32


def _ceil_to(a, b):
    return (a + b - 1) // b * b


def _pack_bf16_pair(c0, c1):
    u0 = jax.lax.bitcast_convert_type(c0, jnp.uint32)
    u1 = jax.lax.bitcast_convert_type(c1, jnp.uint32)
    one = jnp.uint32(1)
    r0 = (u0 + jnp.uint32(0x7FFF) + ((u0 >> 16) & one)) >> 16
    r1 = (u1 + jnp.uint32(0x7FFF) + ((u1 >> 16) & one)) & jnp.uint32(0xFFFF0000)
    return jax.lax.bitcast_convert_type(r0 | r1, jnp.int32)


def _gather_batch(idx_ref, scl_ref, src_ref, K, pi, r0):
    accs = []
    for r in range(_R):
        acc = None
        for k in range(K):
            i = pl.multiple_of(idx_ref[k, r0 + r], pi)
            slab = pltpu.bitcast(src_ref[pl.ds(i, pi), :],
                                 jnp.bfloat16).astype(jnp.float32)
            acc = slab if acc is None else acc + slab
        accs.append(acc * scl_ref[0, 0, r0 + r])
    return accs


def _edge_body(K, t_rows, p, idx_ref, scl_ref, xp_ref, w2_ref, e1_ref, scr):
    pi = p // 2
    for b in range(t_rows // _R):
        r0 = b * _R
        accs = _gather_batch(idx_ref, scl_ref, xp_ref, K, pi, r0)
        for r in range(_R):
            rr = (r0 + r) * p
            scr[pl.ds(rr, p), :] = jnp.maximum(accs[r], 0.0)

    xs = [scr[pl.Slice(j, t_rows, p), :] for j in range(p)]
    xt = jnp.concatenate(xs, axis=-1) if p > 1 else xs[0]
    res = jnp.dot(xt.astype(jnp.bfloat16), w2_ref[...],
                  preferred_element_type=jnp.float32)
    for q in range(pi):
        e1_ref[pl.Slice(q, t_rows, pi), :] = _pack_bf16_pair(
            res[:, (2 * q) * _LANE:(2 * q + 1) * _LANE],
            res[:, (2 * q + 1) * _LANE:(2 * q + 2) * _LANE])


def _node_body(K, t_rows, p, idx_ref, scl_ref, e1_ref, out_ref, scr):
    pi = p // 2
    for b in range(t_rows // _R):
        r0 = b * _R
        accs = _gather_batch(idx_ref, scl_ref, e1_ref, K, pi, r0)
        for r in range(_R):
            rr = (r0 + r) * p
            scr[pl.ds(rr, p), :] = accs[r]

    xs = [scr[pl.Slice(j, t_rows, p), :] for j in range(p)]
    out_ref[...] = jnp.concatenate(xs, axis=-1) if p > 1 else xs[0]


def _xp_body(p, n_real, inter_ref, x_ref, w1_ref, xp_ref):
    i = pl.program_id(0)
    cn = x_ref.shape[0]

    pi = p // 2

    @pl.when(i < n_real)
    def _():
        res = (jnp.dot(x_ref[...].astype(jnp.bfloat16), w1_ref[...],
                       preferred_element_type=jnp.float32)
               * inter_ref[0, 0])
        for q in range(pi):
            xp_ref[pl.Slice(q, cn, pi), :] = _pack_bf16_pair(
                res[:, (2 * q) * _LANE:(2 * q + 1) * _LANE],
                res[:, (2 * q + 1) * _LANE:(2 * q + 2) * _LANE])

    @pl.when(i >= n_real)
    def _():
        xp_ref[...] = jnp.zeros_like(xp_ref)


def _prep_indices(idx, rows_tot, zrow, p):
    R, K = idx.shape
    f32 = jnp.float32
    it = idx.T.astype(jnp.int32)
    mask = it > 0
    cnt = jnp.sum(mask.astype(jnp.int32), axis=0)
    deg = cnt == 0
    idxr = jnp.where(mask, it, zrow)
    idxr = idxr.at[0].set(jnp.where(deg, 0, idxr[0]))
    scl = jnp.where(deg, 1.0, 1.0 / jnp.maximum(cnt, 1).astype(f32)).astype(f32)
    idx_full = jnp.pad(idxr * p, ((0, 0), (0, rows_tot - R)),
                       constant_values=zrow * p)
    scl_full = jnp.pad(scl, (0, rows_tot - R))
    return idx_full, scl_full


def kernel(x, seq, useq, text_vector, w1, w2, w3):
    del text_vector
    f32 = jnp.float32
    bf16 = jnp.bfloat16
    N, Fin = x.shape
    Fout = w1.shape[1]
    E, K1 = seq.shape
    N2, K2 = useq.shape

    F_pad = _ceil_to(Fout, _LANE)
    p = F_pad // _LANE

    w3f = w3.astype(f32)
    tv = w3f[0]
    cosine = (w3f @ tv) / (jnp.linalg.norm(tv) * jnp.linalg.norm(w3f, axis=1))
    inter = jnp.mean(cosine).reshape(1, 1).astype(f32)

    CN = min(512, _ceil_to(N, _R))
    N_pad = _ceil_to(N, CN)
    N_tot = N_pad + CN
    t_e = min(128, _ceil_to(E, _R))
    E_pad = _ceil_to(E, t_e)
    E_tot = E_pad + t_e
    t_n = min(128, _ceil_to(N2, _R))
    N2_pad = _ceil_to(N2, t_n)

    x_p = x if N == N_pad else jnp.pad(x, ((0, N_pad - N), (0, 0)))
    w1_p = jnp.zeros((Fin, F_pad), bf16).at[:, :Fout].set(w1.astype(bf16))
    w2_p = jnp.zeros((F_pad, F_pad), bf16).at[:Fout, :Fout].set(w2.astype(bf16))

    pi = p // 2
    seq_i, seq_s = _prep_indices(seq, E_tot, N_pad, pi)
    useq_i, useq_s = _prep_indices(useq, N2_pad, E_pad, pi)
    n_et = E_tot // t_e
    n_nt = N2_pad // t_n
    seq_s3 = seq_s.reshape(n_et, 1, t_e)
    useq_s3 = useq_s.reshape(n_nt, 1, t_n)

    smem = pltpu.MemorySpace.SMEM
    vmem_limit = 56 * 1024 * 1024

    def resident(shape):
        return pl.BlockSpec(shape, lambda i: tuple(0 for _ in shape),
                            pipeline_mode=pl.Buffered(1))

    n_real = N_pad // CN
    xp2 = pl.pallas_call(
        functools.partial(_xp_body, p, n_real),
        out_shape=jax.ShapeDtypeStruct((N_tot * pi, _LANE), jnp.int32),
        grid=(N_tot // CN,),
        in_specs=[pl.BlockSpec(memory_space=smem),
                  pl.BlockSpec((CN, Fin),
                               lambda i: (jnp.minimum(i, n_real - 1), 0)),
                  resident((Fin, F_pad))],
        out_specs=pl.BlockSpec((CN * pi, _LANE), lambda i: (i, 0)),
        compiler_params=pltpu.CompilerParams(
            dimension_semantics=("parallel",), vmem_limit_bytes=vmem_limit),
    )(inter, x_p, w1_p)

    e12 = pl.pallas_call(
        functools.partial(_edge_body, K1, t_e, p),
        out_shape=jax.ShapeDtypeStruct((E_tot * pi, _LANE), jnp.int32),
        grid=(n_et,),
        in_specs=[
            pl.BlockSpec((K1, t_e), lambda i: (0, i), memory_space=smem),
            pl.BlockSpec((1, 1, t_e), lambda i: (i, 0, 0), memory_space=smem),
            resident((N_tot * pi, _LANE)),
            resident((F_pad, F_pad)),
        ],
        out_specs=pl.BlockSpec((t_e * pi, _LANE), lambda i: (i, 0)),
        scratch_shapes=[pltpu.VMEM((t_e * p, _LANE), f32)],
        compiler_params=pltpu.CompilerParams(
            dimension_semantics=("parallel",), vmem_limit_bytes=vmem_limit),
    )(seq_i, seq_s3, xp2, w2_p)

    node = pl.pallas_call(
        functools.partial(_node_body, K2, t_n, p),
        out_shape=jax.ShapeDtypeStruct((N2_pad, F_pad), f32),
        grid=(n_nt,),
        in_specs=[
            pl.BlockSpec((K2, t_n), lambda i: (0, i), memory_space=smem),
            pl.BlockSpec((1, 1, t_n), lambda i: (i, 0, 0), memory_space=smem),
            resident((E_tot * pi, _LANE)),
        ],
        out_specs=pl.BlockSpec((t_n, F_pad), lambda i: (i, 0)),
        scratch_shapes=[pltpu.VMEM((t_n * p, _LANE), f32)],
        compiler_params=pltpu.CompilerParams(
            dimension_semantics=("parallel",), vmem_limit_bytes=vmem_limit),
    )(useq_i, useq_s3, e12)

    return node[:N2, :Fout]
